# baseline (speedup 1.0000x reference)
import functools
import jax
import jax.numpy as jnp
import numpy as np
from jax.experimental import pallas as pl
from jax.experimental.pallas import tpu as pltpu

N = 10000; E = 320000; E_SEQ = 20000
C_S = 128; C_Z = 64; C_H = 16; H = 8; PQK = 4; PV = 8


def _lin(x, p):
    return x @ p['w'] + p['b']


def _ln(x, p):
    mu = jnp.mean(x, -1, keepdims=True)
    var = jnp.var(x, -1, keepdims=True)
    return (x - mu) / jnp.sqrt(var + 1e-5) * p['g'] + p['b']


def _quat_to_rot(q):
    w, x, y, z = q[..., 0], q[..., 1], q[..., 2], q[..., 3]
    r00 = 1 - 2 * (y * y + z * z); r01 = 2 * (x * y - w * z); r02 = 2 * (x * z + w * y)
    r10 = 2 * (x * y + w * z); r11 = 1 - 2 * (x * x + z * z); r12 = 2 * (y * z - w * x)
    r20 = 2 * (x * z - w * y); r21 = 2 * (y * z + w * x); r22 = 1 - 2 * (x * x + y * y)
    return jnp.stack([jnp.stack([r00, r01, r02], -1), jnp.stack([r10, r11, r12], -1), jnp.stack([r20, r21, r22], -1)], -2)


def _edge_mlp_body(h_ref, w1_ref, b1_ref, w2_ref, b2_ref, w3_ref, b3_ref,
                   g_ref, bn_ref, o_ref):
    h = h_ref[...]
    x = jnp.maximum(jnp.dot(h, w1_ref[...], preferred_element_type=jnp.float32) + b1_ref[...], 0.0)
    x = jnp.maximum(jnp.dot(x, w2_ref[...], preferred_element_type=jnp.float32) + b2_ref[...], 0.0)
    e = jnp.dot(x + h, w3_ref[...], preferred_element_type=jnp.float32) + b3_ref[...]
    mu = jnp.mean(e, -1, keepdims=True)
    var = jnp.mean((e - mu) ** 2, -1, keepdims=True)
    o_ref[...] = (e - mu) * jax.lax.rsqrt(var + 1e-5) * g_ref[...] + bn_ref[...]


@functools.partial(jax.jit, static_argnames=('blk',))
def _edge_mlp(h, w1, b1, w2, b2, w3, b3, g, bn, blk=512):
    ne = h.shape[0]
    pad = (-ne) % blk
    if pad:
        h = jnp.pad(h, ((0, pad), (0, 0)))
    npad = ne + pad
    hid = h.shape[1]
    out = pl.pallas_call(
        _edge_mlp_body,
        grid=(npad // blk,),
        in_specs=[
            pl.BlockSpec((blk, hid), lambda i: (i, 0)),
            pl.BlockSpec((hid, hid), lambda i: (0, 0)),
            pl.BlockSpec((1, hid), lambda i: (0, 0)),
            pl.BlockSpec((hid, hid), lambda i: (0, 0)),
            pl.BlockSpec((1, hid), lambda i: (0, 0)),
            pl.BlockSpec((hid, C_Z), lambda i: (0, 0)),
            pl.BlockSpec((1, C_Z), lambda i: (0, 0)),
            pl.BlockSpec((1, C_Z), lambda i: (0, 0)),
            pl.BlockSpec((1, C_Z), lambda i: (0, 0)),
        ],
        out_specs=pl.BlockSpec((blk, C_Z), lambda i: (i, 0)),
        out_shape=jax.ShapeDtypeStruct((npad, C_Z), jnp.float32),
    )(h, w1, b1[None], w2, b2[None], w3, b3[None], g[None], bn[None])
    return out[:ne]


def _edge_tr(node, edge, edge_index, p):
    ne = _lin(node, p['init'])
    h = jnp.concatenate([edge, ne[edge_index[0]], ne[edge_index[1]]], -1)
    return _edge_mlp(h, p['t1']['w'], p['t1']['b'], p['t2']['w'], p['t2']['b'],
                     p['final']['w'], p['final']['b'], p['ln']['g'], p['ln']['b'])


def _ipa(s, z, edge_index, R, t, mask, p):
    n = s.shape[0]
    src = edge_index[0]
    dst = edge_index[1]
    q = _lin(s, p['wq']).reshape(n, H, C_H)
    kv = _lin(s, p['wkv']).reshape(n, H, 2 * C_H)
    k, v = kv[..., :C_H], kv[..., C_H:]
    qp = _lin(s, p['wqp']).reshape(n, H * PQK, 3)
    qp = jnp.einsum('nij,npj->npi', R, qp) + t[:, None, :]
    qp = qp.reshape(n, H, PQK, 3)
    kvp = _lin(s, p['wkvp']).reshape(n, H * (PQK + PV), 3)
    kvp = jnp.einsum('nij,npj->npi', R, kvp) + t[:, None, :]
    kvp = kvp.reshape(n, H, PQK + PV, 3)
    kp, vp = kvp[:, :, :PQK], kvp[:, :, PQK:]
    b = _lin(z, p['wb'])
    a_sc = jnp.sum(q[dst] * k[src], -1) * (1.0 / np.sqrt(3.0 * C_H))
    d2 = jnp.sum((qp[dst] - kp[src]) ** 2, axis=(-1, -2))
    hw = jax.nn.softplus(p['hw']) * np.sqrt(1.0 / (3.0 * (PQK * 9.0 / 2.0)))
    a = a_sc + b * np.sqrt(1.0 / 3.0) - 0.5 * hw[None, :] * d2
    a = a + 1e9 * (mask[src] - 1.0)[:, None]
    amax = jax.ops.segment_max(a, dst, num_segments=n)
    amax = jnp.where(jnp.isfinite(amax), amax, 0.0)
    ea = jnp.exp(a - amax[dst])
    den = jax.ops.segment_sum(ea, dst, num_segments=n)
    attn = ea / (den[dst] + 1e-9)
    o = jax.ops.segment_sum(attn[:, :, None] * v[src], dst, num_segments=n)
    op = jax.ops.segment_sum(attn[:, :, None, None] * vp[src], dst, num_segments=n)
    op = jnp.einsum('nji,nhpj->nhpi', R, op - t[:, None, None, :])
    opn = jnp.sqrt(jnp.sum(op ** 2, -1) + 1e-8)
    opair = jax.ops.segment_sum(attn[:, :, None] * z[:, None, :], dst, num_segments=n)
    feat = jnp.concatenate([o.reshape(n, -1), op.reshape(n, -1), opn.reshape(n, -1), opair.reshape(n, -1)], -1)
    return _lin(feat, p['wo'])


def _node_tr(s, p):
    x = jax.nn.relu(_lin(s, p['l1']))
    x = jax.nn.relu(_lin(x, p['l2']))
    x = _lin(x, p['l3'])
    return _ln(s + x, p['ln'])


def _compose(R, t, upd):
    quat = jnp.concatenate([jnp.ones_like(upd[:, :1]), upd[:, :3]], -1)
    quat = quat / jnp.linalg.norm(quat, axis=-1, keepdims=True)
    Rq = _quat_to_rot(quat)
    Rn = jnp.einsum('nij,njk->nik', R, Rq)
    tn = t + jnp.einsum('nij,nj->ni', R, upd[:, 3:])
    return Rn, tn


def kernel(node_features, rots, trans, edge_features, edge_index, seq_edge_features, seq_edge_index, res_mask, noising_mask, params):
    m = res_mask
    s = node_features
    u = _ipa(s, edge_features, edge_index, rots, trans, m, params['ipa_sp'])
    s = _ln(s + u * m[:, None], params['ln1'])
    u = _ipa(s, seq_edge_features, seq_edge_index, rots, trans, m, params['ipa_sq'])
    s = _ln(s + u * m[:, None], params['ln2'])
    s = _node_tr(s, params['nt'])
    s = s * m[:, None]
    upd = _lin(s * noising_mask[:, None], params['bb']) * noising_mask[:, None]
    rn, tn = _compose(rots, trans, upd)
    e = _edge_tr(s, edge_features, edge_index, params['et'])
    se = _edge_tr(s, seq_edge_features, seq_edge_index, params['set'])
    return s, rn, tn, e, se


# speedup vs baseline: 1.0956x; 1.0956x over previous
import functools
import jax
import jax.numpy as jnp
import numpy as np
from jax import lax
from jax.experimental import pallas as pl
from jax.experimental.pallas import tpu as pltpu
from jax.experimental.pallas import tpu_sc as plsc

N = 10000; E = 320000; E_SEQ = 20000
C_S = 128; C_Z = 64; C_H = 16; H = 8; PQK = 4; PV = 8

_NW = 32
_GCH = 256


@functools.partial(jax.jit, static_argnames=('d',))
def _sc_gather_p(table, idx, d):
    b = idx.shape[0]
    b_per_w = b // _NW
    n_ch = b_per_w // _GCH
    mesh = plsc.VectorSubcoreMesh(core_axis_name="c", subcore_axis_name="s")

    @functools.partial(
        pl.kernel, mesh=mesh,
        out_type=jax.ShapeDtypeStruct((b, d), jnp.float32),
        scratch_types=[pltpu.VMEM((_GCH,), jnp.int32),
                       pltpu.VMEM((_GCH, d), jnp.float32),
                       pltpu.SemaphoreType.DMA],
    )
    def k(table_hbm, idx_hbm, out_hbm, idx_v, rows_v, sem):
        wid = lax.axis_index("s") * 2 + lax.axis_index("c")
        base = wid * b_per_w

        def body(j, carry):
            off = base + j * _GCH
            pltpu.sync_copy(idx_hbm.at[pl.ds(off, _GCH)], idx_v)
            pltpu.async_copy(table_hbm.at[idx_v], rows_v, sem).wait()
            pltpu.sync_copy(rows_v, out_hbm.at[pl.ds(off, _GCH)])
            return carry

        lax.fori_loop(0, n_ch, body, 0)

    return k(table, idx)


def _sc_gather(table, idx, n_real):
    d = table.shape[1]
    dpad = (-d) % 128
    if dpad:
        table = jnp.pad(table, ((0, 0), (0, dpad)))
    out = _sc_gather_p(table, idx, d + dpad)
    return out[:n_real, :d]


def _pad_idx(idx):
    step = _NW * _GCH
    b = idx.shape[0]
    pad = (-b) % step
    if pad:
        idx = jnp.pad(idx, (0, pad))
    return idx


def _lin(x, p):
    return x @ p['w'] + p['b']


def _ln(x, p):
    mu = jnp.mean(x, -1, keepdims=True)
    var = jnp.var(x, -1, keepdims=True)
    return (x - mu) / jnp.sqrt(var + 1e-5) * p['g'] + p['b']


def _quat_to_rot(q):
    w, x, y, z = q[..., 0], q[..., 1], q[..., 2], q[..., 3]
    r00 = 1 - 2 * (y * y + z * z); r01 = 2 * (x * y - w * z); r02 = 2 * (x * z + w * y)
    r10 = 2 * (x * y + w * z); r11 = 1 - 2 * (x * x + z * z); r12 = 2 * (y * z - w * x)
    r20 = 2 * (x * z - w * y); r21 = 2 * (y * z + w * x); r22 = 1 - 2 * (x * x + y * y)
    return jnp.stack([jnp.stack([r00, r01, r02], -1), jnp.stack([r10, r11, r12], -1), jnp.stack([r20, r21, r22], -1)], -2)


def _edge_mlp_body(h_ref, w1_ref, b1_ref, w2_ref, b2_ref, w3_ref, b3_ref,
                   g_ref, bn_ref, o_ref):
    h = h_ref[...]
    x = jnp.maximum(jnp.dot(h, w1_ref[...], preferred_element_type=jnp.float32) + b1_ref[...], 0.0)
    x = jnp.maximum(jnp.dot(x, w2_ref[...], preferred_element_type=jnp.float32) + b2_ref[...], 0.0)
    e = jnp.dot(x + h, w3_ref[...], preferred_element_type=jnp.float32) + b3_ref[...]
    mu = jnp.mean(e, -1, keepdims=True)
    var = jnp.mean((e - mu) ** 2, -1, keepdims=True)
    o_ref[...] = (e - mu) * jax.lax.rsqrt(var + 1e-5) * g_ref[...] + bn_ref[...]


@functools.partial(jax.jit, static_argnames=('blk',))
def _edge_mlp(h, w1, b1, w2, b2, w3, b3, g, bn, blk=512):
    ne = h.shape[0]
    pad = (-ne) % blk
    if pad:
        h = jnp.pad(h, ((0, pad), (0, 0)))
    npad = ne + pad
    hid = h.shape[1]
    out = pl.pallas_call(
        _edge_mlp_body,
        grid=(npad // blk,),
        in_specs=[
            pl.BlockSpec((blk, hid), lambda i: (i, 0)),
            pl.BlockSpec((hid, hid), lambda i: (0, 0)),
            pl.BlockSpec((1, hid), lambda i: (0, 0)),
            pl.BlockSpec((hid, hid), lambda i: (0, 0)),
            pl.BlockSpec((1, hid), lambda i: (0, 0)),
            pl.BlockSpec((hid, C_Z), lambda i: (0, 0)),
            pl.BlockSpec((1, C_Z), lambda i: (0, 0)),
            pl.BlockSpec((1, C_Z), lambda i: (0, 0)),
            pl.BlockSpec((1, C_Z), lambda i: (0, 0)),
        ],
        out_specs=pl.BlockSpec((blk, C_Z), lambda i: (i, 0)),
        out_shape=jax.ShapeDtypeStruct((npad, C_Z), jnp.float32),
    )(h, w1, b1[None], w2, b2[None], w3, b3[None], g[None], bn[None])
    return out[:ne]


def _edge_tr(node, edge, src_pad, dst_pad, p):
    ne = _lin(node, p['init'])
    e_real = edge.shape[0]
    h = jnp.concatenate([edge, _sc_gather(ne, src_pad, e_real),
                         _sc_gather(ne, dst_pad, e_real)], -1)
    return _edge_mlp(h, p['t1']['w'], p['t1']['b'], p['t2']['w'], p['t2']['b'],
                     p['final']['w'], p['final']['b'], p['ln']['g'], p['ln']['b'])


def _ipa(s, z, edge_index, src_pad, dst_pad, R, t, mask, p):
    n = s.shape[0]
    ne = edge_index.shape[1]
    src = edge_index[0]
    dst = edge_index[1]
    q = _lin(s, p['wq'])
    kv = _lin(s, p['wkv']).reshape(n, H, 2 * C_H)
    k = kv[..., :C_H].reshape(n, H * C_H)
    v = kv[..., C_H:].reshape(n, H * C_H)
    qp = _lin(s, p['wqp']).reshape(n, H * PQK, 3)
    qp = (jnp.einsum('nij,npj->npi', R, qp) + t[:, None, :]).reshape(n, H * PQK * 3)
    kvp = _lin(s, p['wkvp']).reshape(n, H * (PQK + PV), 3)
    kvp = jnp.einsum('nij,npj->npi', R, kvp) + t[:, None, :]
    kvp = kvp.reshape(n, H, PQK + PV, 3)
    kp = kvp[:, :, :PQK].reshape(n, H * PQK * 3)
    vp = kvp[:, :, PQK:].reshape(n, H * PV * 3)
    b = _lin(z, p['wb'])
    qd = _sc_gather(q, dst_pad, ne).reshape(ne, H, C_H)
    ks = _sc_gather(k, src_pad, ne).reshape(ne, H, C_H)
    qpd = _sc_gather(qp, dst_pad, ne).reshape(ne, H, PQK, 3)
    kps = _sc_gather(kp, src_pad, ne).reshape(ne, H, PQK, 3)
    a_sc = jnp.sum(qd * ks, -1) * (1.0 / np.sqrt(3.0 * C_H))
    d2 = jnp.sum((qpd - kps) ** 2, axis=(-1, -2))
    hw = jax.nn.softplus(p['hw']) * np.sqrt(1.0 / (3.0 * (PQK * 9.0 / 2.0)))
    a = a_sc + b * np.sqrt(1.0 / 3.0) - 0.5 * hw[None, :] * d2
    a = a + 1e9 * (mask[src] - 1.0)[:, None]
    amax = jax.ops.segment_max(a, dst, num_segments=n)
    amax = jnp.where(jnp.isfinite(amax), amax, 0.0)
    ea = jnp.exp(a - _sc_gather(amax, dst_pad, ne))
    den = jax.ops.segment_sum(ea, dst, num_segments=n)
    inv_den = 1.0 / (den + 1e-9)
    vs = _sc_gather(v, src_pad, ne).reshape(ne, H, C_H)
    vps = _sc_gather(vp, src_pad, ne).reshape(ne, H, PV, 3)
    o = jax.ops.segment_sum(ea[:, :, None] * vs, dst, num_segments=n) * inv_den[:, :, None]
    op = jax.ops.segment_sum(ea[:, :, None, None] * vps, dst, num_segments=n) * inv_den[:, :, None, None]
    op = jnp.einsum('nji,nhpj->nhpi', R, op - t[:, None, None, :])
    opn = jnp.sqrt(jnp.sum(op ** 2, -1) + 1e-8)
    opair = jax.ops.segment_sum(ea[:, :, None] * z[:, None, :], dst, num_segments=n) * inv_den[:, :, None]
    feat = jnp.concatenate([o.reshape(n, -1), op.reshape(n, -1), opn.reshape(n, -1), opair.reshape(n, -1)], -1)
    return _lin(feat, p['wo'])


def _node_tr(s, p):
    x = jax.nn.relu(_lin(s, p['l1']))
    x = jax.nn.relu(_lin(x, p['l2']))
    x = _lin(x, p['l3'])
    return _ln(s + x, p['ln'])


def _compose(R, t, upd):
    quat = jnp.concatenate([jnp.ones_like(upd[:, :1]), upd[:, :3]], -1)
    quat = quat / jnp.linalg.norm(quat, axis=-1, keepdims=True)
    Rq = _quat_to_rot(quat)
    Rn = jnp.einsum('nij,njk->nik', R, Rq)
    tn = t + jnp.einsum('nij,nj->ni', R, upd[:, 3:])
    return Rn, tn


def kernel(node_features, rots, trans, edge_features, edge_index, seq_edge_features, seq_edge_index, res_mask, noising_mask, params):
    m = res_mask
    s = node_features
    src_pad = _pad_idx(edge_index[0])
    dst_pad = _pad_idx(edge_index[1])
    ssrc_pad = _pad_idx(seq_edge_index[0])
    sdst_pad = _pad_idx(seq_edge_index[1])
    u = _ipa(s, edge_features, edge_index, src_pad, dst_pad, rots, trans, m, params['ipa_sp'])
    s = _ln(s + u * m[:, None], params['ln1'])
    u = _ipa(s, seq_edge_features, seq_edge_index, ssrc_pad, sdst_pad, rots, trans, m, params['ipa_sq'])
    s = _ln(s + u * m[:, None], params['ln2'])
    s = _node_tr(s, params['nt'])
    s = s * m[:, None]
    upd = _lin(s * noising_mask[:, None], params['bb']) * noising_mask[:, None]
    rn, tn = _compose(rots, trans, upd)
    e = _edge_tr(s, edge_features, src_pad, dst_pad, params['et'])
    se = _edge_tr(s, seq_edge_features, ssrc_pad, sdst_pad, params['set'])
    return s, rn, tn, e, se


# speedup vs baseline: 2.6477x; 2.4167x over previous
import functools
import jax
import jax.numpy as jnp
import numpy as np
from jax import lax
from jax.experimental import pallas as pl
from jax.experimental.pallas import tpu as pltpu
from jax.experimental.pallas import tpu_sc as plsc

N = 10000; E = 320000; E_SEQ = 20000
C_S = 128; C_Z = 64; C_H = 16; H = 8; PQK = 4; PV = 8

_NW = 32
_GCH = 256


@functools.partial(jax.jit, static_argnames=('d',))
def _sc_gather_p(table, idx, d):
    b = idx.shape[0]
    b_per_w = b // _NW
    n_ch = b_per_w // _GCH
    mesh = plsc.VectorSubcoreMesh(core_axis_name="c", subcore_axis_name="s")

    @functools.partial(
        pl.kernel, mesh=mesh,
        out_type=jax.ShapeDtypeStruct((b, d), jnp.float32),
        scratch_types=[pltpu.VMEM((_GCH,), jnp.int32),
                       pltpu.VMEM((_GCH, d), jnp.float32),
                       pltpu.SemaphoreType.DMA],
    )
    def k(table_hbm, idx_hbm, out_hbm, idx_v, rows_v, sem):
        wid = lax.axis_index("s") * 2 + lax.axis_index("c")
        base = wid * b_per_w

        def body(j, carry):
            off = base + j * _GCH
            pltpu.sync_copy(idx_hbm.at[pl.ds(off, _GCH)], idx_v)
            pltpu.async_copy(table_hbm.at[idx_v], rows_v, sem).wait()
            pltpu.sync_copy(rows_v, out_hbm.at[pl.ds(off, _GCH)])
            return carry

        lax.fori_loop(0, n_ch, body, 0)

    return k(table, idx)


def _sc_gather(table, idx, n_real):
    d = table.shape[1]
    dpad = (-d) % 128
    if dpad:
        table = jnp.pad(table, ((0, 0), (0, dpad)))
    out = _sc_gather_p(table, idx, d + dpad)
    return out[:n_real, :d]


def _pad_idx(idx):
    step = _NW * _GCH
    b = idx.shape[0]
    pad = (-b) % step
    if pad:
        idx = jnp.pad(idx, (0, pad))
    return idx


_SCH = 128


@functools.partial(jax.jit, static_argnames=('d',))
def _sc_segsum_p(vals, dst_idx, zeros, d):
    b = dst_idx.shape[0]
    b_per_w = b // _NW
    n_ch = b_per_w // _SCH
    nz = N // _SCH
    mesh = plsc.VectorSubcoreMesh(core_axis_name="c", subcore_axis_name="s")

    @functools.partial(
        pl.kernel, mesh=mesh,
        out_type=jax.ShapeDtypeStruct((2, N, d), jnp.float32),
        scratch_types=[pltpu.VMEM((_SCH,), jnp.int32),
                       pltpu.VMEM((_SCH, d), jnp.float32),
                       pltpu.VMEM_SHARED((N, d), jnp.float32)],
    )
    def k(vals_hbm, dst_hbm, zeros_hbm, out_hbm, idx_v, rows_v, acc_sh):
        cid = lax.axis_index("c")
        sid = lax.axis_index("s")
        wid = sid * 2 + cid
        base = wid * b_per_w

        @pl.when(sid == 0)
        def _zero():
            def zchunk(j, c):
                pltpu.sync_copy(zeros_hbm.at[pl.ds(j * _SCH, _SCH)], rows_v)
                pltpu.sync_copy(rows_v, acc_sh.at[pl.ds(j * _SCH, _SCH)])
                return c
            lax.fori_loop(0, nz, zchunk, 0)
            pltpu.sync_copy(zeros_hbm.at[pl.ds(0, 16)],
                            rows_v.at[pl.ds(0, 16)])
            pltpu.sync_copy(rows_v.at[pl.ds(0, 16)],
                            acc_sh.at[pl.ds(nz * _SCH, 16)])

        plsc.subcore_barrier()

        def chunk(j, c):
            off = base + j * _SCH
            pltpu.sync_copy(dst_hbm.at[pl.ds(off, _SCH)], idx_v)
            pltpu.sync_copy(vals_hbm.at[pl.ds(off, _SCH)], rows_v)
            pltpu.sync_copy(rows_v, acc_sh.at[idx_v], add=True)
            return c
        lax.fori_loop(0, n_ch, chunk, 0)

        plsc.subcore_barrier()

        @pl.when(sid == 0)
        def _dump():
            def dchunk(j, c):
                pltpu.sync_copy(acc_sh.at[pl.ds(j * _SCH, _SCH)], rows_v)
                pltpu.sync_copy(rows_v, out_hbm.at[cid, pl.ds(j * _SCH, _SCH)])
                return c
            lax.fori_loop(0, nz, dchunk, 0)
            pltpu.sync_copy(acc_sh.at[pl.ds(nz * _SCH, 16)],
                            rows_v.at[pl.ds(0, 16)])
            pltpu.sync_copy(rows_v.at[pl.ds(0, 16)],
                            out_hbm.at[cid, pl.ds(nz * _SCH, 16)])

    return k(vals, dst_idx, zeros)


def _sc_segsum(vals, dst_pad):
    e_real, d = vals.shape
    pad = dst_pad.shape[0] - e_real
    if pad:
        vals = jnp.pad(vals, ((0, pad), (0, 0)))
    dpad = (-d) % 128
    if dpad:
        vals = jnp.pad(vals, ((0, 0), (0, dpad)))
    zeros = jnp.zeros((N, 128), jnp.float32)
    outs = []
    for c0 in range(0, d + dpad, 128):
        part = _sc_segsum_p(vals[:, c0:c0 + 128], dst_pad, zeros, 128)
        outs.append(part[0] + part[1])
    return jnp.concatenate(outs, -1)[:, :d]


def _sc_segmax_p(a_flat, dst_idx):
    b = dst_idx.shape[0]
    e_per_w = b // _NW
    n_ch = e_per_w // _GCH
    n8 = N * H
    mesh = plsc.VectorSubcoreMesh(core_axis_name="c", subcore_axis_name="s")

    @functools.partial(
        pl.kernel, mesh=mesh,
        out_type=jax.ShapeDtypeStruct((_NW, n8), jnp.float32),
        scratch_types=[pltpu.VMEM((n8 + 16,), jnp.float32),
                       pltpu.VMEM((_GCH,), jnp.int32),
                       pltpu.VMEM((_GCH * H + 16,), jnp.float32)],
    )
    def k(a_hbm, dst_hbm, out_hbm, acc_v, idx_v, a_v):
        wid = lax.axis_index("s") * 2 + lax.axis_index("c")
        base = wid * e_per_w
        ninf = jnp.full((16,), -jnp.inf, jnp.float32)

        def init(j, c):
            acc_v[pl.ds(j * 16, 16)] = ninf
            return c
        lax.fori_loop(0, n8 // 16 + 1, init, 0)

        lane = lax.iota(jnp.int32, 16)
        msk = lane < H

        def chunk(j, c):
            off = base + j * _GCH
            pltpu.sync_copy(dst_hbm.at[pl.ds(off, _GCH)], idx_v)
            pltpu.sync_copy(a_hbm.at[pl.ds(off * H, _GCH * H)],
                            a_v.at[pl.ds(0, _GCH * H)])

            def edge16(i, c2):
                dstv = idx_v[pl.ds(i * 16, 16)]
                for l in range(16):
                    d = dstv[l]
                    off2 = pl.multiple_of(d * H, 8)
                    aval = jnp.where(msk, a_v[pl.ds(i * 128 + l * H, 16)],
                                     ninf)
                    cur = acc_v[pl.ds(off2, 16)]
                    acc_v[pl.ds(off2, 16)] = jnp.maximum(cur, aval)
                return c2
            lax.fori_loop(0, _GCH // 16, edge16, 0)
            return c
        lax.fori_loop(0, n_ch, chunk, 0)
        pltpu.sync_copy(acc_v.at[pl.ds(0, n8)], out_hbm.at[wid])

    return k(a_flat, dst_idx)


def _sc_segmax(a, dst_pad):
    e_real = a.shape[0]
    pad = dst_pad.shape[0] - e_real
    af = jnp.pad(a, ((0, pad), (0, 0)), constant_values=-jnp.inf)
    part = _sc_segmax_p(af.reshape(-1), dst_pad)
    return jnp.max(part.reshape(_NW, N, H), axis=0)


def _lin(x, p):
    return x @ p['w'] + p['b']


def _ln(x, p):
    mu = jnp.mean(x, -1, keepdims=True)
    var = jnp.var(x, -1, keepdims=True)
    return (x - mu) / jnp.sqrt(var + 1e-5) * p['g'] + p['b']


def _quat_to_rot(q):
    w, x, y, z = q[..., 0], q[..., 1], q[..., 2], q[..., 3]
    r00 = 1 - 2 * (y * y + z * z); r01 = 2 * (x * y - w * z); r02 = 2 * (x * z + w * y)
    r10 = 2 * (x * y + w * z); r11 = 1 - 2 * (x * x + z * z); r12 = 2 * (y * z - w * x)
    r20 = 2 * (x * z - w * y); r21 = 2 * (y * z + w * x); r22 = 1 - 2 * (x * x + y * y)
    return jnp.stack([jnp.stack([r00, r01, r02], -1), jnp.stack([r10, r11, r12], -1), jnp.stack([r20, r21, r22], -1)], -2)


def _edge_mlp_body(h_ref, w1_ref, b1_ref, w2_ref, b2_ref, w3_ref, b3_ref,
                   g_ref, bn_ref, o_ref):
    h = h_ref[...]
    x = jnp.maximum(jnp.dot(h, w1_ref[...], preferred_element_type=jnp.float32) + b1_ref[...], 0.0)
    x = jnp.maximum(jnp.dot(x, w2_ref[...], preferred_element_type=jnp.float32) + b2_ref[...], 0.0)
    e = jnp.dot(x + h, w3_ref[...], preferred_element_type=jnp.float32) + b3_ref[...]
    mu = jnp.mean(e, -1, keepdims=True)
    var = jnp.mean((e - mu) ** 2, -1, keepdims=True)
    o_ref[...] = (e - mu) * jax.lax.rsqrt(var + 1e-5) * g_ref[...] + bn_ref[...]


@functools.partial(jax.jit, static_argnames=('blk',))
def _edge_mlp(h, w1, b1, w2, b2, w3, b3, g, bn, blk=512):
    ne = h.shape[0]
    pad = (-ne) % blk
    if pad:
        h = jnp.pad(h, ((0, pad), (0, 0)))
    npad = ne + pad
    hid = h.shape[1]
    out = pl.pallas_call(
        _edge_mlp_body,
        grid=(npad // blk,),
        in_specs=[
            pl.BlockSpec((blk, hid), lambda i: (i, 0)),
            pl.BlockSpec((hid, hid), lambda i: (0, 0)),
            pl.BlockSpec((1, hid), lambda i: (0, 0)),
            pl.BlockSpec((hid, hid), lambda i: (0, 0)),
            pl.BlockSpec((1, hid), lambda i: (0, 0)),
            pl.BlockSpec((hid, C_Z), lambda i: (0, 0)),
            pl.BlockSpec((1, C_Z), lambda i: (0, 0)),
            pl.BlockSpec((1, C_Z), lambda i: (0, 0)),
            pl.BlockSpec((1, C_Z), lambda i: (0, 0)),
        ],
        out_specs=pl.BlockSpec((blk, C_Z), lambda i: (i, 0)),
        out_shape=jax.ShapeDtypeStruct((npad, C_Z), jnp.float32),
    )(h, w1, b1[None], w2, b2[None], w3, b3[None], g[None], bn[None])
    return out[:ne]


def _edge_tr(node, edge, src_pad, dst_pad, p):
    ne = _lin(node, p['init'])
    e_real = edge.shape[0]
    h = jnp.concatenate([edge, _sc_gather(ne, src_pad, e_real),
                         _sc_gather(ne, dst_pad, e_real)], -1)
    return _edge_mlp(h, p['t1']['w'], p['t1']['b'], p['t2']['w'], p['t2']['b'],
                     p['final']['w'], p['final']['b'], p['ln']['g'], p['ln']['b'])


def _ipa(s, z, edge_index, src_pad, dst_pad, R, t, mask, p):
    n = s.shape[0]
    ne = edge_index.shape[1]
    src = edge_index[0]
    dst = edge_index[1]
    q = _lin(s, p['wq'])
    kv = _lin(s, p['wkv']).reshape(n, H, 2 * C_H)
    k = kv[..., :C_H].reshape(n, H * C_H)
    v = kv[..., C_H:].reshape(n, H * C_H)
    qp = _lin(s, p['wqp']).reshape(n, H * PQK, 3)
    qp = (jnp.einsum('nij,npj->npi', R, qp) + t[:, None, :]).reshape(n, H * PQK * 3)
    kvp = _lin(s, p['wkvp']).reshape(n, H * (PQK + PV), 3)
    kvp = jnp.einsum('nij,npj->npi', R, kvp) + t[:, None, :]
    kvp = kvp.reshape(n, H, PQK + PV, 3)
    kp = kvp[:, :, :PQK].reshape(n, H * PQK * 3)
    vp = kvp[:, :, PQK:].reshape(n, H * PV * 3)
    b = _lin(z, p['wb'])
    qd = _sc_gather(q, dst_pad, ne).reshape(ne, H, C_H)
    ks = _sc_gather(k, src_pad, ne).reshape(ne, H, C_H)
    qpd = _sc_gather(qp, dst_pad, ne).reshape(ne, H, PQK, 3)
    kps = _sc_gather(kp, src_pad, ne).reshape(ne, H, PQK, 3)
    a_sc = jnp.sum(qd * ks, -1) * (1.0 / np.sqrt(3.0 * C_H))
    d2 = jnp.sum((qpd - kps) ** 2, axis=(-1, -2))
    hw = jax.nn.softplus(p['hw']) * np.sqrt(1.0 / (3.0 * (PQK * 9.0 / 2.0)))
    a = a_sc + b * np.sqrt(1.0 / 3.0) - 0.5 * hw[None, :] * d2
    a = a + 1e9 * (mask[src] - 1.0)[:, None]
    amax = _sc_segmax(a, dst_pad)
    amax = jnp.where(jnp.isfinite(amax), amax, 0.0)
    ea = jnp.exp(a - _sc_gather(amax, dst_pad, ne))
    vs = _sc_gather(v, src_pad, ne).reshape(ne, H, C_H)
    vps = _sc_gather(vp, src_pad, ne).reshape(ne, H, PV, 3)
    orows = jnp.concatenate([(ea[:, :, None] * vs).reshape(ne, H * C_H),
                             ea, jnp.zeros((ne, 8), jnp.float32)], -1)
    osum = _sc_segsum(orows, dst_pad)
    den = osum[:, H * C_H:H * C_H + H]
    inv_den = 1.0 / (den + 1e-9)
    o = osum[:, :H * C_H].reshape(n, H, C_H) * inv_den[:, :, None]
    oprows = (ea[:, :, None, None] * vps).reshape(ne, H * PV * 3)
    op = _sc_segsum(oprows, dst_pad).reshape(n, H, PV, 3) * inv_den[:, :, None, None]
    op = jnp.einsum('nji,nhpj->nhpi', R, op - t[:, None, None, :])
    opn = jnp.sqrt(jnp.sum(op ** 2, -1) + 1e-8)
    opair = jax.ops.segment_sum(ea[:, :, None] * z[:, None, :], dst, num_segments=n) * inv_den[:, :, None]
    feat = jnp.concatenate([o.reshape(n, -1), op.reshape(n, -1), opn.reshape(n, -1), opair.reshape(n, -1)], -1)
    return _lin(feat, p['wo'])


def _node_tr(s, p):
    x = jax.nn.relu(_lin(s, p['l1']))
    x = jax.nn.relu(_lin(x, p['l2']))
    x = _lin(x, p['l3'])
    return _ln(s + x, p['ln'])


def _compose(R, t, upd):
    quat = jnp.concatenate([jnp.ones_like(upd[:, :1]), upd[:, :3]], -1)
    quat = quat / jnp.linalg.norm(quat, axis=-1, keepdims=True)
    Rq = _quat_to_rot(quat)
    Rn = jnp.einsum('nij,njk->nik', R, Rq)
    tn = t + jnp.einsum('nij,nj->ni', R, upd[:, 3:])
    return Rn, tn


def kernel(node_features, rots, trans, edge_features, edge_index, seq_edge_features, seq_edge_index, res_mask, noising_mask, params):
    m = res_mask
    s = node_features
    src_pad = _pad_idx(edge_index[0])
    dst_pad = _pad_idx(edge_index[1])
    ssrc_pad = _pad_idx(seq_edge_index[0])
    sdst_pad = _pad_idx(seq_edge_index[1])
    u = _ipa(s, edge_features, edge_index, src_pad, dst_pad, rots, trans, m, params['ipa_sp'])
    s = _ln(s + u * m[:, None], params['ln1'])
    u = _ipa(s, seq_edge_features, seq_edge_index, ssrc_pad, sdst_pad, rots, trans, m, params['ipa_sq'])
    s = _ln(s + u * m[:, None], params['ln2'])
    s = _node_tr(s, params['nt'])
    s = s * m[:, None]
    upd = _lin(s * noising_mask[:, None], params['bb']) * noising_mask[:, None]
    rn, tn = _compose(rots, trans, upd)
    e = _edge_tr(s, edge_features, src_pad, dst_pad, params['et'])
    se = _edge_tr(s, seq_edge_features, ssrc_pad, sdst_pad, params['set'])
    return s, rn, tn, e, se
